# speedup vs baseline: 4.6238x; 4.6238x over previous
import functools

import jax
import jax.numpy as jnp
from jax import lax
from jax.experimental import pallas as pl
from jax.experimental.pallas import tpu as pltpu
from jax.experimental.pallas import tpu_sc as plsc

D = 128
R = 1024
CH = 128
LANES = 16


def _dense_in_body(x_ref, wr_ref, b_ref, wo_ref, y_ref, r_ref):
    xb = x_ref[...]
    y_ref[...] = jnp.dot(xb, wr_ref[...], preferred_element_type=jnp.float32)
    r_ref[...] = jnp.dot(xb, wo_ref[...], preferred_element_type=jnp.float32) + b_ref[...]


def _dense_mid_body(n_real, acc_ref, r1_ref, wr_ref, b_ref, wo_ref, y2_ref, r2_ref):
    h = jnp.maximum(acc_ref[0] + acc_ref[1] + r1_ref[...], 0.0)
    rows = pl.program_id(0) * R + lax.broadcasted_iota(jnp.int32, (R, 1), 0)
    h = jnp.where(rows < n_real, h, 0.0)
    y2_ref[...] = jnp.dot(h, wr_ref[...], preferred_element_type=jnp.float32)
    r2_ref[...] = jnp.dot(h, wo_ref[...], preferred_element_type=jnp.float32) + b_ref[...]


def _dense_out_body(acc_ref, r2_ref, o_ref):
    o_ref[...] = acc_ref[0] + acc_ref[1] + r2_ref[...]


def _row_spec():
    return pl.BlockSpec((R, D), lambda i: (i, 0))


def _w_spec():
    return pl.BlockSpec((D, D), lambda i: (0, 0))


def _b_spec():
    return pl.BlockSpec((1, D), lambda i: (0, 0))


def _acc_spec():
    return pl.BlockSpec((2, R, D), lambda i: (0, i, 0))


def _dense_in(xp, wr, b, wo, np_):
    return pl.pallas_call(
        _dense_in_body,
        grid=(np_ // R,),
        in_specs=[_row_spec(), _w_spec(), _b_spec(), _w_spec()],
        out_specs=[_row_spec(), _row_spec()],
        out_shape=[jax.ShapeDtypeStruct((np_, D), jnp.float32)] * 2,
    )(xp, wr, b, wo)


def _dense_mid(acc, r1, wr, b, wo, np_, n):
    return pl.pallas_call(
        functools.partial(_dense_mid_body, n),
        grid=(np_ // R,),
        in_specs=[_acc_spec(), _row_spec(), _w_spec(), _b_spec(), _w_spec()],
        out_specs=[_row_spec(), _row_spec()],
        out_shape=[jax.ShapeDtypeStruct((np_, D), jnp.float32)] * 2,
    )(acc, r1, wr, b, wo)


def _dense_out(acc, r2, np_):
    return pl.pallas_call(
        _dense_out_body,
        grid=(np_ // R,),
        in_specs=[_acc_spec(), _row_spec()],
        out_specs=_row_spec(),
        out_shape=jax.ShapeDtypeStruct((np_, D), jnp.float32),
    )(acc, r2)


@functools.lru_cache(maxsize=None)
def _make_segsum(np_, k, nc, ns):
    rpt = np_ // ns
    mesh = plsc.VectorSubcoreMesh(core_axis_name="c", subcore_axis_name="s")

    @functools.partial(
        pl.kernel,
        mesh=mesh,
        out_type=jax.ShapeDtypeStruct((nc, np_, D), jnp.float32),
        scratch_types=[
            pltpu.VMEM((k, CH), jnp.int32),
            pltpu.VMEM((k, CH), jnp.int32),
            pltpu.VMEM((CH, D), jnp.float32),
            pltpu.VMEM_SHARED((np_, D), jnp.float32),
            pltpu.SemaphoreType.DMA,
        ],
    )
    def seg(y_hbm, src_hbm, dst_hbm, out_hbm, src_v, dst_v, rows_v, acc_sh, sem):
        c = lax.axis_index("c")
        s = lax.axis_index("s")
        wid = c * ns + s

        def zrow(i, carry):
            for j in range(D // LANES):
                rows_v[i, pl.ds(j * LANES, LANES)] = jnp.zeros((LANES,), jnp.float32)
            return carry
        lax.fori_loop(0, CH, zrow, 0)
        for t in range(rpt // CH):
            pltpu.sync_copy(rows_v, acc_sh.at[pl.ds(s * rpt + t * CH, CH)])
        plsc.subcore_barrier()

        pltpu.sync_copy(src_hbm.at[wid], src_v)
        pltpu.sync_copy(dst_hbm.at[wid], dst_v)

        def step(j, carry):
            pltpu.async_copy(y_hbm.at[src_v.at[j]], rows_v, sem).wait()
            pltpu.sync_copy(rows_v, acc_sh.at[dst_v.at[j]], add=True)
            return carry
        lax.fori_loop(0, k, step, 0)
        plsc.subcore_barrier()

        for t in range(rpt // CH):
            sl = pl.ds(s * rpt + t * CH, CH)
            pltpu.sync_copy(acc_sh.at[sl], out_hbm.at[c, sl])

    return seg


def kernel(x, edge_index, W1_rel, b1, W1_root, W2_rel, b2, W2_root):
    n, d = x.shape
    e = edge_index.shape[1]
    assert d == D
    try:
        info = plsc.get_sparse_core_info()
        nc, ns = info.num_cores, info.num_subcores
    except Exception:
        nc, ns = 2, 16
    nw = nc * ns
    align = max(ns * CH, R)
    np_ = -(-(n + 1) // align) * align
    k = -(-e // (nw * CH))
    ep = nw * k * CH

    src = edge_index[0]
    dst = edge_index[1]
    srcp = jnp.full((ep,), n, jnp.int32).at[:e].set(src).reshape(nw, k, CH)
    dstp = jnp.full((ep,), n, jnp.int32).at[:e].set(dst).reshape(nw, k, CH)
    xp = jnp.pad(x, ((0, np_ - n), (0, 0)))
    b1r = b1.reshape(1, D)
    b2r = b2.reshape(1, D)

    seg = _make_segsum(np_, k, nc, ns)
    y1, r1 = _dense_in(xp, W1_rel, b1r, W1_root, np_)
    acc1 = seg(y1, srcp, dstp)
    y2, r2 = _dense_mid(acc1, r1, W2_rel, b2r, W2_root, np_, n)
    acc2 = seg(y2, srcp, dstp)
    outp = _dense_out(acc2, r2, np_)
    return outp[:n]
